# baseline (speedup 1.0000x reference)
import functools

import jax
import jax.numpy as jnp
import numpy as np
from jax import lax
from jax.experimental import pallas as pl
from jax.experimental.pallas import tpu as pltpu
from jax.experimental.pallas import tpu_sc as plsc

_EPS = 1e-10
_ALPHA = 0.8
_K = 9
_N = 20000
_BN = 5000
_NBLK = 4
_GP = 128
_BIGI = 1 << 30
_NSC = 20000


_NOISE_CACHE = {}


def _noise_arrays(num_gt):
    if num_gt in _NOISE_CACHE:
        return _NOISE_CACHE[num_gt]
    try:
        with jax.default_device(jax.devices("cpu")[0]):
            u = jax.random.uniform(jax.random.key(1), (num_gt, _K, 2),
                                   dtype=jnp.float32)
            noise = np.asarray((u - 0.5) * 0.1)
        nxh = np.zeros((16, _GP), np.float32)
        nyh = np.zeros((16, _GP), np.float32)
        nxh[:_K, :num_gt] = noise[:, :, 0].T
        nyh[:_K, :num_gt] = noise[:, :, 1].T
        _NOISE_CACHE[num_gt] = (nxh, nyh)
        return nxh, nyh
    except Exception:
        u = jax.random.uniform(jax.random.key(1), (num_gt, _K, 2),
                               dtype=jnp.float32)
        noise = (u - 0.5) * 0.1
        nx = jnp.zeros((16, _GP), jnp.float32).at[:_K, :num_gt].set(
            noise[:, :, 0].T)
        ny = jnp.zeros((16, _GP), jnp.float32).at[:_K, :num_gt].set(
            noise[:, :, 1].T)
        return nx, ny


def _score_topk_kernel(preds_ref, bb_ref, cls_ref, oh_ref, gt_ref,
                       vals_ref, gidx_ref, cxt_ref, cyt_ref):
    i = pl.program_id(0)
    px1 = preds_ref[:, 0:1]
    py1 = preds_ref[:, 1:2]
    px2 = preds_ref[:, 2:3]
    py2 = preds_ref[:, 3:4]
    area1 = (px2 - px1) * (py2 - py1)

    gx1 = gt_ref[0:1, :]
    gy1 = gt_ref[1:2, :]
    gx2 = gt_ref[2:3, :]
    gy2 = gt_ref[3:4, :]
    area2 = (gx2 - gx1) * (gy2 - gy1)

    ltx = jnp.maximum(px1, gx1)
    lty = jnp.maximum(py1, gy1)
    rbx = jnp.minimum(px2, gx2)
    rby = jnp.minimum(py2, gy2)
    inter = jnp.clip(rbx - ltx, 0.0, None) * jnp.clip(rby - lty, 0.0, None)
    union = jnp.maximum(area1 + area2 - inter, 1e-6)
    iou = inter / union
    ov_pow = jnp.where(iou > 0.0, jnp.maximum(iou, _EPS) ** _ALPHA, 0.0)

    cls_sel = jnp.dot(cls_ref[...], oh_ref[...],
                      preferred_element_type=jnp.float32)
    sig = 1.0 / (1.0 + jnp.exp(-cls_sel))
    scores = sig ** (1.0 - _ALPHA) * ov_pow

    cxt_ref[...] = (bb_ref[:, 0:1] + bb_ref[:, 2:3]) * 0.5
    cyt_ref[...] = (bb_ref[:, 1:2] + bb_ref[:, 3:4]) * 0.5

    riota = jax.lax.broadcasted_iota(jnp.int32, (_BN, _GP), 0)

    v_rows, i_rows = [], []
    for _ in range(_K):
        m = jnp.max(scores, axis=0, keepdims=True)
        lidx = jnp.min(jnp.where(scores == m, riota, _BN),
                       axis=0, keepdims=True)
        msk = riota == lidx
        v_rows.append(m)
        i_rows.append(lidx + i * _BN)
        scores = jnp.where(msk, -1.0, scores)

    pad_f = jnp.full((16 - _K, _GP), -1.0, jnp.float32)
    pad_i = jnp.full((16 - _K, _GP), _BIGI, jnp.int32)
    vals_ref[...] = jnp.concatenate(v_rows + [pad_f], axis=0)
    gidx_ref[...] = jnp.concatenate(i_rows + [pad_i], axis=0)


def _merge_kernel(vals_ref, gidx_ref, fidx_ref):
    vals = vals_ref[...]
    gidx = gidx_ref[...]

    i_rows = []
    for _ in range(_K):
        m = jnp.max(vals, axis=0, keepdims=True)
        g = jnp.min(jnp.where(vals == m, gidx, _BIGI),
                    axis=0, keepdims=True)
        msk = gidx == g
        i_rows.append(g)
        vals = jnp.where(msk, -1.0, vals)

    pad_i = jnp.zeros((16 - _K, _GP), jnp.int32)
    fidx_ref[...] = jnp.concatenate(i_rows + [pad_i], axis=0)


def _shift16(x, offsets):
    dn = lax.GatherDimensionNumbers(
        offset_dims=(), collapsed_slice_dims=(0,), start_index_map=(0,))
    return lax.gather(x, offsets[:, None], dn, (1,),
                      mode=lax.GatherScatterMode.PROMISE_IN_BOUNDS)


def _sc_tail_body(num_gt,
                  fidx_hbm, cxt_hbm, cyt_hbm,
                  nx_hbm, ny_hbm, gt_hbm,
                  out_hbm,
                  img, idxv, cxv, cyv, wvv, gtv, nxv, nyv,
                  gsem):
    cid = lax.axis_index("c")
    sid = lax.axis_index("s")

    @pl.when(jnp.logical_and(cid == 0, sid == 0))
    def _():
        pltpu.sync_copy(fidx_hbm, idxv)
        pltpu.sync_copy(nx_hbm, nxv)
        pltpu.sync_copy(ny_hbm, nyv)
        pltpu.sync_copy(gt_hbm, gtv)

        copies = []
        for j in range(_K):
            copies.append(pltpu.async_copy(
                cxt_hbm.at[idxv.at[j]], cxv.at[j], gsem))
            copies.append(pltpu.async_copy(
                cyt_hbm.at[idxv.at[j]], cyv.at[j], gsem))

        def zero_body(i, carry):
            img[pl.ds(i * 16, 16)] = jnp.zeros((16,), jnp.float32)
            return carry

        lax.fori_loop(0, _NSC // 16, zero_body, 0, unroll=4)

        for c in copies:
            c.wait()

        lane = lax.iota(jnp.int32, 16)
        up_off = jnp.minimum(lane + 1, 15)
        dn_offs = [jnp.maximum(lane - s, 0) for s in (1, 2, 4, 8)]

        for b in range(_GP // 16):
            gcol0 = b * 16
            fcx = [cxv[k, gcol0:gcol0 + 16] for k in range(_K)]
            fcy = [cyv[k, gcol0:gcol0 + 16] for k in range(_K)]
            dx = [fcx[k] + nxv[k, gcol0:gcol0 + 16] for k in range(_K)]
            dy = [fcy[k] + nyv[k, gcol0:gcol0 + 16] for k in range(_K)]
            inv_k = 1.0 / _K
            miu_x = sum(dx) * inv_k
            miu_y = sum(dy) * inv_k
            dxn = [v - miu_x for v in dx]
            dyn = [v - miu_y for v in dy]
            sxx = sum(v * v for v in dxn) * inv_k
            sxy = sum(a * b2 for a, b2 in zip(dxn, dyn)) * inv_k
            syy = sum(v * v for v in dyn) * inv_k
            det = sxx * syy - sxy * sxy
            denom = det + 1e-10
            i00 = syy / denom
            i01 = -sxy / denom
            i11 = sxx / denom

            gx1 = gtv[0, gcol0:gcol0 + 16]
            gy1 = gtv[1, gcol0:gcol0 + 16]
            gx2 = gtv[2, gcol0:gcol0 + 16]
            gy2 = gtv[3, gcol0:gcol0 + 16]
            gmask = (lane + gcol0) < num_gt

            for k in range(_K):
                dxc = fcx[k] - miu_x
                dyc = fcy[k] - miu_y
                t0 = dxc * i00 + dyc * i01
                t1 = dxc * i01 + dyc * i11
                quad = t0 * dxc + t1 * dyc
                wgt = jnp.exp(-0.5 * quad)
                valid = ((fcx[k] - gx1 > _EPS) & (fcy[k] - gy1 > _EPS)
                         & (gx2 - fcx[k] > _EPS) & (gy2 - fcy[k] > _EPS))
                wv = jnp.where(valid & gmask, wgt, 0.0)
                wvv[k, gcol0:gcol0 + 16] = wv

        for k in range(_K):
            for c in range(_GP // 16):
                aidx = idxv[k, c * 16:(c + 1) * 16]
                wv = wvv[k, c * 16:(c + 1) * 16]
                key, wv = plsc.sort_key_val(aidx, wv)
                for si, off in zip((1, 2, 4, 8), dn_offs):
                    pk = _shift16(key, off)
                    pw = _shift16(wv, off)
                    same = (pk == key) & (lane >= si)
                    wv = jnp.where(same, jnp.maximum(wv, pw), wv)
                nk = _shift16(key, up_off)
                is_end = (nk != key) | (lane == 15)
                cur = plsc.load_gather(img, [key])
                nv = jnp.maximum(cur, wv)
                plsc.store_scatter(img, [key], nv, mask=is_end)

        pltpu.sync_copy(img, out_hbm)


def kernel(bboxes, cls_scores, bbox_preds, gt_bboxes, bbox_levels, gt_labels):
    f32 = cls_scores.dtype
    N, C = cls_scores.shape
    G = gt_bboxes.shape[0]

    labels_pad = jnp.full((_GP,), -1, jnp.int32).at[:G].set(
        gt_labels.astype(jnp.int32))
    onehot = (labels_pad[None, :]
              == jnp.arange(C, dtype=jnp.int32)[:, None]).astype(f32)
    gt_cmp = jnp.zeros((8, _GP), f32).at[:4, :G].set(gt_bboxes.T)

    nxa, nya = _noise_arrays(G)
    nx = jnp.asarray(nxa, f32)
    ny = jnp.asarray(nya, f32)

    vals, gidx, cxt, cyt = pl.pallas_call(
        _score_topk_kernel,
        grid=(_NBLK,),
        in_specs=[
            pl.BlockSpec((_BN, 4), lambda i: (i, 0)),
            pl.BlockSpec((_BN, 4), lambda i: (i, 0)),
            pl.BlockSpec((_BN, C), lambda i: (i, 0)),
            pl.BlockSpec((C, _GP), lambda i: (0, 0)),
            pl.BlockSpec((8, _GP), lambda i: (0, 0)),
        ],
        out_specs=[
            pl.BlockSpec((16, _GP), lambda i: (i, 0)),
            pl.BlockSpec((16, _GP), lambda i: (i, 0)),
            pl.BlockSpec((_BN, 1), lambda i: (i, 0)),
            pl.BlockSpec((_BN, 1), lambda i: (i, 0)),
        ],
        out_shape=[
            jax.ShapeDtypeStruct((_NBLK * 16, _GP), jnp.float32),
            jax.ShapeDtypeStruct((_NBLK * 16, _GP), jnp.int32),
            jax.ShapeDtypeStruct((N, 1), jnp.float32),
            jax.ShapeDtypeStruct((N, 1), jnp.float32),
        ],
    )(bbox_preds[:, :4], bboxes[:, :4], cls_scores, onehot, gt_cmp)

    fidx = pl.pallas_call(
        _merge_kernel,
        out_shape=jax.ShapeDtypeStruct((16, _GP), jnp.int32),
    )(vals, gidx)

    mesh = plsc.VectorSubcoreMesh(core_axis_name="c", subcore_axis_name="s")
    sc_tail = functools.partial(
        pl.kernel,
        mesh=mesh,
        out_type=jax.ShapeDtypeStruct((_NSC,), jnp.float32),
        scratch_types=[
            pltpu.VMEM((_NSC,), jnp.float32),
            pltpu.VMEM((16, _GP), jnp.int32),
            pltpu.VMEM((16, _GP), jnp.float32),
            pltpu.VMEM((16, _GP), jnp.float32),
            pltpu.VMEM((16, _GP), jnp.float32),
            pltpu.VMEM((8, _GP), jnp.float32),
            pltpu.VMEM((16, _GP), jnp.float32),
            pltpu.VMEM((16, _GP), jnp.float32),
            pltpu.SemaphoreType.DMA,
        ],
        compiler_params=pltpu.CompilerParams(needs_layout_passes=False),
    )(functools.partial(_sc_tail_body, G))
    out = sc_tail(fidx, cxt.reshape(N), cyt.reshape(N), nx, ny, gt_cmp)

    return out[:N].astype(f32)


# speedup vs baseline: 1.2665x; 1.2665x over previous
import functools

import jax
import jax.numpy as jnp
import numpy as np
from jax import lax
from jax.experimental import pallas as pl
from jax.experimental.pallas import tpu as pltpu
from jax.experimental.pallas import tpu_sc as plsc

_EPS = 1e-10
_ALPHA = 0.8
_K = 9
_N = 20000
_BN = 5000
_NBLK = 4
_GP = 128
_BIGI = 1 << 30
_NSC = 20000


_NOISE_CACHE = {}


def _noise_arrays(num_gt):
    if num_gt in _NOISE_CACHE:
        return _NOISE_CACHE[num_gt]
    try:
        with jax.default_device(jax.devices("cpu")[0]):
            u = jax.random.uniform(jax.random.key(1), (num_gt, _K, 2),
                                   dtype=jnp.float32)
            noise = np.asarray((u - 0.5) * 0.1)
        nxh = np.zeros((16, _GP), np.float32)
        nyh = np.zeros((16, _GP), np.float32)
        nxh[:_K, :num_gt] = noise[:, :, 0].T
        nyh[:_K, :num_gt] = noise[:, :, 1].T
        _NOISE_CACHE[num_gt] = (nxh, nyh)
        return nxh, nyh
    except Exception:
        u = jax.random.uniform(jax.random.key(1), (num_gt, _K, 2),
                               dtype=jnp.float32)
        noise = (u - 0.5) * 0.1
        nx = jnp.zeros((16, _GP), jnp.float32).at[:_K, :num_gt].set(
            noise[:, :, 0].T)
        ny = jnp.zeros((16, _GP), jnp.float32).at[:_K, :num_gt].set(
            noise[:, :, 1].T)
        return nx, ny


def _score_topk_kernel(preds_ref, cls_ref, oh_ref, gt_ref,
                       fidx_ref, svals_ref, sgidx_ref):
    i = pl.program_id(0)
    px1 = preds_ref[:, 0:1]
    py1 = preds_ref[:, 1:2]
    px2 = preds_ref[:, 2:3]
    py2 = preds_ref[:, 3:4]
    area1 = (px2 - px1) * (py2 - py1)

    gx1 = gt_ref[0:1, :]
    gy1 = gt_ref[1:2, :]
    gx2 = gt_ref[2:3, :]
    gy2 = gt_ref[3:4, :]
    area2 = (gx2 - gx1) * (gy2 - gy1)

    ltx = jnp.maximum(px1, gx1)
    lty = jnp.maximum(py1, gy1)
    rbx = jnp.minimum(px2, gx2)
    rby = jnp.minimum(py2, gy2)
    inter = jnp.clip(rbx - ltx, 0.0, None) * jnp.clip(rby - lty, 0.0, None)
    union = jnp.maximum(area1 + area2 - inter, 1e-6)
    iou = inter / union
    ov_pow = jnp.where(iou > 0.0, jnp.maximum(iou, _EPS) ** _ALPHA, 0.0)

    cls_sel = jnp.dot(cls_ref[...], oh_ref[...],
                      preferred_element_type=jnp.float32)
    sig = 1.0 / (1.0 + jnp.exp(-cls_sel))
    scores = sig ** (1.0 - _ALPHA) * ov_pow

    riota = jax.lax.broadcasted_iota(jnp.int32, (_BN, _GP), 0)

    v_rows, i_rows = [], []
    for _ in range(_K):
        m = jnp.max(scores, axis=0, keepdims=True)
        lidx = jnp.min(jnp.where(scores == m, riota, _BN),
                       axis=0, keepdims=True)
        msk = riota == lidx
        v_rows.append(m)
        i_rows.append(lidx + i * _BN)
        scores = jnp.where(msk, -1.0, scores)

    pad_f = jnp.full((16 - _K, _GP), -1.0, jnp.float32)
    pad_i = jnp.full((16 - _K, _GP), _BIGI, jnp.int32)
    svals_ref[pl.ds(i * 16, 16), :] = jnp.concatenate(v_rows + [pad_f], axis=0)
    sgidx_ref[pl.ds(i * 16, 16), :] = jnp.concatenate(i_rows + [pad_i], axis=0)

    @pl.when(i == _NBLK - 1)
    def _():
        vals = svals_ref[...]
        gidx = sgidx_ref[...]
        f_rows = []
        mv = vals
        for _ in range(_K):
            m = jnp.max(mv, axis=0, keepdims=True)
            g = jnp.min(jnp.where(mv == m, gidx, _BIGI),
                        axis=0, keepdims=True)
            msk = gidx == g
            f_rows.append(g)
            mv = jnp.where(msk, -1.0, mv)
        fpad = jnp.zeros((16 - _K, _GP), jnp.int32)
        fidx_ref[...] = jnp.concatenate(f_rows + [fpad], axis=0)


def _shift16(x, offsets):
    dn = lax.GatherDimensionNumbers(
        offset_dims=(), collapsed_slice_dims=(0,), start_index_map=(0,))
    return lax.gather(x, offsets[:, None], dn, (1,),
                      mode=lax.GatherScatterMode.PROMISE_IN_BOUNDS)


def _sc_tail_body(num_gt,
                  fidx_hbm, bx1_hbm, by1_hbm, bx2_hbm, by2_hbm,
                  nx_hbm, ny_hbm, gt_hbm,
                  out_hbm,
                  img, idxv, cx1v, cx2v, cy1v, cy2v, wvv, gtv, nxv, nyv,
                  gsem):
    cid = lax.axis_index("c")
    sid = lax.axis_index("s")

    @pl.when(jnp.logical_and(cid == 0, sid == 0))
    def _():
        pltpu.sync_copy(fidx_hbm, idxv)
        pltpu.sync_copy(nx_hbm, nxv)
        pltpu.sync_copy(ny_hbm, nyv)
        pltpu.sync_copy(gt_hbm, gtv)

        copies = []
        for j in range(_K):
            copies.append(pltpu.async_copy(
                bx1_hbm.at[idxv.at[j]], cx1v.at[j], gsem))
            copies.append(pltpu.async_copy(
                bx2_hbm.at[idxv.at[j]], cx2v.at[j], gsem))
            copies.append(pltpu.async_copy(
                by1_hbm.at[idxv.at[j]], cy1v.at[j], gsem))
            copies.append(pltpu.async_copy(
                by2_hbm.at[idxv.at[j]], cy2v.at[j], gsem))

        def zero_body(i, carry):
            img[pl.ds(i * 16, 16)] = jnp.zeros((16,), jnp.float32)
            return carry

        lax.fori_loop(0, _NSC // 16, zero_body, 0, unroll=4)

        for c in copies:
            c.wait()

        lane = lax.iota(jnp.int32, 16)
        up_off = jnp.minimum(lane + 1, 15)
        dn_offs = [jnp.maximum(lane - s, 0) for s in (1, 2, 4, 8)]

        for b in range(_GP // 16):
            gcol0 = b * 16
            fcx = [(cx1v[k, gcol0:gcol0 + 16]
                    + cx2v[k, gcol0:gcol0 + 16]) * 0.5 for k in range(_K)]
            fcy = [(cy1v[k, gcol0:gcol0 + 16]
                    + cy2v[k, gcol0:gcol0 + 16]) * 0.5 for k in range(_K)]
            dx = [fcx[k] + nxv[k, gcol0:gcol0 + 16] for k in range(_K)]
            dy = [fcy[k] + nyv[k, gcol0:gcol0 + 16] for k in range(_K)]
            inv_k = 1.0 / _K
            miu_x = sum(dx) * inv_k
            miu_y = sum(dy) * inv_k
            dxn = [v - miu_x for v in dx]
            dyn = [v - miu_y for v in dy]
            sxx = sum(v * v for v in dxn) * inv_k
            sxy = sum(a * b2 for a, b2 in zip(dxn, dyn)) * inv_k
            syy = sum(v * v for v in dyn) * inv_k
            det = sxx * syy - sxy * sxy
            denom = det + 1e-10
            i00 = syy / denom
            i01 = -sxy / denom
            i11 = sxx / denom

            gx1 = gtv[0, gcol0:gcol0 + 16]
            gy1 = gtv[1, gcol0:gcol0 + 16]
            gx2 = gtv[2, gcol0:gcol0 + 16]
            gy2 = gtv[3, gcol0:gcol0 + 16]
            gmask = (lane + gcol0) < num_gt

            for k in range(_K):
                dxc = fcx[k] - miu_x
                dyc = fcy[k] - miu_y
                t0 = dxc * i00 + dyc * i01
                t1 = dxc * i01 + dyc * i11
                quad = t0 * dxc + t1 * dyc
                wgt = jnp.exp(-0.5 * quad)
                valid = ((fcx[k] - gx1 > _EPS) & (fcy[k] - gy1 > _EPS)
                         & (gx2 - fcx[k] > _EPS) & (gy2 - fcy[k] > _EPS))
                wv = jnp.where(valid & gmask, wgt, 0.0)
                wvv[k, gcol0:gcol0 + 16] = wv

        for k in range(_K):
            for c in range(_GP // 16):
                aidx = idxv[k, c * 16:(c + 1) * 16]
                wv = wvv[k, c * 16:(c + 1) * 16]
                key, wv = plsc.sort_key_val(aidx, wv)
                for si, off in zip((1, 2, 4, 8), dn_offs):
                    pk = _shift16(key, off)
                    pw = _shift16(wv, off)
                    same = (pk == key) & (lane >= si)
                    wv = jnp.where(same, jnp.maximum(wv, pw), wv)
                nk = _shift16(key, up_off)
                is_end = (nk != key) | (lane == 15)
                cur = plsc.load_gather(img, [key])
                nv = jnp.maximum(cur, wv)
                plsc.store_scatter(img, [key], nv, mask=is_end)

        pltpu.sync_copy(img, out_hbm)


def kernel(bboxes, cls_scores, bbox_preds, gt_bboxes, bbox_levels, gt_labels):
    f32 = cls_scores.dtype
    N, C = cls_scores.shape
    G = gt_bboxes.shape[0]

    labels_pad = jnp.full((_GP,), -1, jnp.int32).at[:G].set(
        gt_labels.astype(jnp.int32))
    onehot = (labels_pad[None, :]
              == jnp.arange(C, dtype=jnp.int32)[:, None]).astype(f32)
    gt_cmp = jnp.zeros((8, _GP), f32).at[:4, :G].set(gt_bboxes.T)

    nxa, nya = _noise_arrays(G)
    nx = jnp.asarray(nxa, f32)
    ny = jnp.asarray(nya, f32)

    fidx = pl.pallas_call(
        _score_topk_kernel,
        grid=(_NBLK,),
        in_specs=[
            pl.BlockSpec((_BN, 4), lambda i: (i, 0)),
            pl.BlockSpec((_BN, C), lambda i: (i, 0)),
            pl.BlockSpec((C, _GP), lambda i: (0, 0)),
            pl.BlockSpec((8, _GP), lambda i: (0, 0)),
        ],
        out_specs=pl.BlockSpec((16, _GP), lambda i: (0, 0)),
        out_shape=jax.ShapeDtypeStruct((16, _GP), jnp.int32),
        scratch_shapes=[
            pltpu.VMEM((_NBLK * 16, _GP), jnp.float32),
            pltpu.VMEM((_NBLK * 16, _GP), jnp.int32),
        ],
    )(bbox_preds[:, :4], cls_scores, onehot, gt_cmp)

    mesh = plsc.VectorSubcoreMesh(core_axis_name="c", subcore_axis_name="s")
    sc_tail = functools.partial(
        pl.kernel,
        mesh=mesh,
        out_type=jax.ShapeDtypeStruct((_NSC,), jnp.float32),
        scratch_types=[
            pltpu.VMEM((_NSC,), jnp.float32),
            pltpu.VMEM((16, _GP), jnp.int32),
            pltpu.VMEM((16, _GP), jnp.float32),
            pltpu.VMEM((16, _GP), jnp.float32),
            pltpu.VMEM((16, _GP), jnp.float32),
            pltpu.VMEM((16, _GP), jnp.float32),
            pltpu.VMEM((16, _GP), jnp.float32),
            pltpu.VMEM((8, _GP), jnp.float32),
            pltpu.VMEM((16, _GP), jnp.float32),
            pltpu.VMEM((16, _GP), jnp.float32),
            pltpu.SemaphoreType.DMA,
        ],
        compiler_params=pltpu.CompilerParams(needs_layout_passes=False),
    )(functools.partial(_sc_tail_body, G))
    out = sc_tail(fidx,
                  jnp.asarray(bboxes[:, 0], f32),
                  jnp.asarray(bboxes[:, 1], f32),
                  jnp.asarray(bboxes[:, 2], f32),
                  jnp.asarray(bboxes[:, 3], f32),
                  nx, ny, gt_cmp)

    return out[:N].astype(f32)
